# baseline (speedup 1.0000x reference)
import jax
import jax.numpy as jnp
from jax import lax
from jax.experimental import pallas as pl
from jax.experimental.pallas import tpu as pltpu
from jax.experimental.pallas import tpu_sc as plsc

NUM_ROWS = 1000000
D = 64
BATCH = 16384

NC = 2
NS = 16
NW = NC * NS
L = 16
DC = D // L
CW = 128
NCH = NUM_ROWS // CW
TAIL = NUM_ROWS - NCH * CW
BASE_CH = NCH // NW
EXTRA = NCH - BASE_CH * NW
WCAP = 8192
DUMP = 2 * BATCH
NB = BATCH // NW


def _lane_sum16(v):
    io = lax.iota(jnp.int32, L)
    dn = lax.GatherDimensionNumbers(
        offset_dims=(), collapsed_slice_dims=(0,), start_index_map=(0,))
    for k in (8, 4, 2, 1):
        idx = lax.bitwise_xor(io, jnp.full((L,), k, dtype=jnp.int32))
        v = v + lax.gather(v, idx[:, None], dn, (1,),
                           mode=lax.GatherScatterMode.PROMISE_IN_BOUNDS)
    return v


def _rsqrt16(x):
    i = plsc.bitcast(x, jnp.int32)
    y = plsc.bitcast(
        jnp.full((L,), 0x5F3759DF, dtype=jnp.int32)
        - lax.shift_right_logical(i, jnp.full((L,), 1, dtype=jnp.int32)),
        jnp.float32,
    )
    half = x * 0.5
    for _ in range(3):
        y = y * (1.5 - half * y * y)
    return y


def _gather_body(left_hbm, right_hbm, table_t, stage_hbm,
                 idx_v, wl_r, wl_j, cl_r, cl_j, chunk, colblk, tailb,
                 semc, sems):
    wid = lax.axis_index("s") * NC + lax.axis_index("c")
    off = wid * BASE_CH + jnp.minimum(wid, EXTRA)
    nch = BASE_CH + jnp.where(wid < EXTRA, 1, 0)
    lo = off * CW
    hi = jnp.where(wid == NW - 1, NUM_ROWS, (off + nch) * CW)

    pltpu.sync_copy(left_hbm, idx_v.at[0])
    pltpu.sync_copy(right_hbm, idx_v.at[1])

    def scan_side(side, total0):
        def scan(i, tot):
            v = idx_v[side, pl.ds(i * L, L)]
            m = (v >= lo) & (v < hi)
            mi = jnp.where(m, jnp.int32(1), jnp.int32(0))
            pos = jnp.minimum(plsc.cumsum(mi) - 1 + tot, WCAP - 1)
            slot = lax.iota(jnp.int32, L) + (i * L + side * BATCH)
            plsc.store_scatter(wl_r, [pos], v - lo, mask=m)
            plsc.store_scatter(wl_j, [pos], slot, mask=m)
            return tot + plsc.all_reduce_population_count(m)[0]
        return lax.fori_loop(0, BATCH // L, scan, total0)

    total = scan_side(0, jnp.int32(0))
    total = scan_side(1, total)
    total = jnp.minimum(total, WCAP)
    nwl = (total + L - 1) // L

    def fire(g):
        cb = (off + g) * CW
        pltpu.make_async_copy(
            table_t.at[:, pl.ds(cb * 1, CW)], chunk.at[lax.rem(g, 2)],
            semc).start()

    fire(0)

    def do_chunk(g, carry):
        buf = lax.rem(g, 2)
        pltpu.make_async_copy(
            table_t.at[:, pl.ds(0, CW)], chunk.at[buf], semc).wait()

        @pl.when(g < nch - 1)
        def _():
            fire(g + 1)

        cb_loc = (off + g) * CW - lo

        def coll(q, cnt):
            rv = wl_r[pl.ds(q * L, L)]
            jv = wl_j[pl.ds(q * L, L)]
            lane = lax.iota(jnp.int32, L) + q * L
            lv = rv - cb_loc
            m = (lane < total) & (lv >= 0) & (lv < CW)
            mi = jnp.where(m, jnp.int32(1), jnp.int32(0))
            pos = jnp.minimum(plsc.cumsum(mi) - 1 + cnt, WCAP - 1)
            plsc.store_scatter(cl_r, [pos], lv, mask=m)
            plsc.store_scatter(cl_j, [pos], jv, mask=m)
            return cnt + plsc.all_reduce_population_count(m)[0]

        cnt = lax.fori_loop(0, nwl, coll, jnp.int32(0))

        def extract(t, carry2):
            rv = cl_r[pl.ds(t * L, L)]
            jv = cl_j[pl.ds(t * L, L)]
            lane = lax.iota(jnp.int32, L) + t * L
            m = lane < cnt
            rs = jnp.where(m, rv, 0)
            js = jnp.where(m, jv, DUMP)
            cbuf = lax.rem(t, 2)
            for c in range(D):
                vals = plsc.load_gather(
                    chunk.at[buf], [jnp.full((L,), c, jnp.int32), rs])
                plsc.store_scatter(
                    colblk.at[cbuf],
                    [lax.iota(jnp.int32, L), jnp.full((L,), c, jnp.int32)],
                    vals)
            cp = pltpu.make_async_copy(
                stage_hbm.at[js], colblk.at[cbuf], sems)
            cp.start()
            cp.wait()
            return carry2

        lax.fori_loop(0, (cnt + L - 1) // L, extract, 0)
        return carry

    lax.fori_loop(0, nch, do_chunk, 0)

    @pl.when(wid == NW - 1)
    def _tail():
        for gq in range(8):
            pltpu.sync_copy(
                table_t.at[pl.ds(gq * 8, 8), pl.ds(NCH * CW, TAIL)],
                tailb.at[gq])

        def tail_entry(q, carry):
            rv = wl_r[pl.ds(0, L)]
            return carry

        tb = NCH * CW - lo

        def tscan(q, carry):
            rv = wl_r[pl.ds(q * L, L)]
            jv = wl_j[pl.ds(q * L, L)]
            lane = lax.iota(jnp.int32, L) + q * L
            lv = rv - tb
            m = (lane < total) & (lv >= 0)
            mi32 = jnp.where(m, jnp.int32(1), jnp.int32(0))

            def one(k, carry2):
                @pl.when(mi32[k] == 1)
                def _():
                    kk = lv[k]
                    khi = (kk // L) * L
                    klo = kk - khi
                    dn = lax.GatherDimensionNumbers(
                        offset_dims=(), collapsed_slice_dims=(0,),
                        start_index_map=(0,))
                    for c in range(D):
                        row = tailb[c // 8, c % 8, pl.ds(khi, L)]
                        val = lax.gather(
                            row, jnp.full((L, 1), klo, jnp.int32), dn, (1,),
                            mode=lax.GatherScatterMode.PROMISE_IN_BOUNDS)
                        plsc.store_scatter(
                            colblk.at[0],
                            [jnp.zeros((L,), jnp.int32),
                             jnp.full((L,), c, jnp.int32)],
                            val, mask=lax.iota(jnp.int32, L) < 1)
                    js = jnp.where(lax.iota(jnp.int32, L) < 1,
                                   jnp.full((L,), jv[k], jnp.int32), DUMP)
                    cp = pltpu.make_async_copy(
                        stage_hbm.at[js], colblk.at[0], sems)
                    cp.start()
                    cp.wait()
                return carry2

            for k in range(L):
                one(k, 0)
            return carry

        lax.fori_loop(0, nwl, tscan, 0)


def _norm_body(stage_hbm, pos_l_hbm, pos_r_hbm, out_hbm,
               sl_v, sr_v, out_v, pos_v, sem):
    wid = lax.axis_index("s") * NC + lax.axis_index("c")
    base = wid * NB

    pltpu.sync_copy(pos_l_hbm, pos_v.at[0])
    pltpu.sync_copy(pos_r_hbm, pos_v.at[1])
    psum = [pos_v[0, pl.ds(c * L, L)] + pos_v[1, pl.ds(c * L, L)]
            for c in range(DC)]

    half = NB // 2
    for h in range(2):
        hbase = base + h * half
        pltpu.sync_copy(stage_hbm.at[pl.ds(hbase, half)], sl_v)
        pltpu.sync_copy(stage_hbm.at[pl.ds(BATCH + hbase, half)], sr_v)

        def row(i, carry):
            s = [sl_v[i, pl.ds(c * L, L)] + sr_v[i, pl.ds(c * L, L)]
                 + psum[c] for c in range(DC)]
            ss = s[0] * s[0]
            for c in range(1, DC):
                ss = ss + s[c] * s[c]
            rinv = _rsqrt16(jnp.maximum(_lane_sum16(ss), 1e-24))
            gi = h * half + i
            row2 = gi // 2
            offc = lax.rem(gi, 2) * D
            for c in range(DC):
                out_v[row2, pl.ds(offc + c * L, L)] = s[c] * rinv
            return carry

        lax.fori_loop(0, half, row, 0)

    pltpu.sync_copy(out_v, out_hbm.at[pl.ds(wid * (NB // 2), NB // 2)])


@jax.jit
def kernel(left_idx, right_idx, class_emb, pos_left, pos_right):
    mesh = plsc.VectorSubcoreMesh(core_axis_name="c", subcore_axis_name="s")
    cp = pltpu.CompilerParams(
        needs_layout_passes=False, use_tc_tiling_on_sc=True)

    gather = pl.kernel(
        _gather_body,
        out_type=jax.ShapeDtypeStruct((2 * BATCH + 1, CW), jnp.float32),
        mesh=mesh,
        compiler_params=cp,
        scratch_types=[
            pltpu.VMEM((2, BATCH), jnp.int32),
            pltpu.VMEM((WCAP,), jnp.int32),
            pltpu.VMEM((WCAP,), jnp.int32),
            pltpu.VMEM((WCAP,), jnp.int32),
            pltpu.VMEM((WCAP,), jnp.int32),
            pltpu.VMEM((2, D, CW), jnp.float32),
            pltpu.VMEM((2, L, CW), jnp.float32),
            pltpu.VMEM((8, 8, TAIL), jnp.float32),
            pltpu.SemaphoreType.DMA,
            pltpu.SemaphoreType.DMA,
        ],
    )

    norm = pl.kernel(
        _norm_body,
        out_type=jax.ShapeDtypeStruct((BATCH // 2, 2 * D), jnp.float32),
        mesh=mesh,
        compiler_params=cp,
        scratch_types=[
            pltpu.VMEM((BATCH // NW // 2, CW), jnp.float32),
            pltpu.VMEM((BATCH // NW // 2, CW), jnp.float32),
            pltpu.VMEM((BATCH // NW // 2, 2 * D), jnp.float32),
            pltpu.VMEM((2, D), jnp.float32),
            pltpu.SemaphoreType.DMA,
        ],
    )

    li = left_idx.astype(jnp.int32)
    ri = right_idx.astype(jnp.int32)
    stage = gather(li, ri, class_emb.T)
    out2 = norm(stage, pos_left, pos_right)
    return out2.reshape(BATCH, D)


# speedup vs baseline: 8.4268x; 8.4268x over previous
import jax
import jax.numpy as jnp
from jax import lax
from jax.experimental import pallas as pl
from jax.experimental.pallas import tpu as pltpu
from jax.experimental.pallas import tpu_sc as plsc

NUM_ROWS = 1000000
D = 64
BATCH = 16384

NC = 2
NS = 16
NW = NC * NS
B_PER_W = BATCH // NW
L = 16
DC = D // L
NG = B_PER_W // L


def _lane_sum16(v):
    io = lax.iota(jnp.int32, L)
    dn = lax.GatherDimensionNumbers(
        offset_dims=(), collapsed_slice_dims=(0,), start_index_map=(0,))
    for k in (8, 4, 2, 1):
        idx = lax.bitwise_xor(io, jnp.full((L,), k, dtype=jnp.int32))
        v = v + lax.gather(v, idx[:, None], dn, (1,),
                           mode=lax.GatherScatterMode.PROMISE_IN_BOUNDS)
    return v


def _rsqrt16(x):
    i = plsc.bitcast(x, jnp.int32)
    y = plsc.bitcast(
        jnp.full((L,), 0x5F3759DF, dtype=jnp.int32)
        - lax.shift_right_logical(i, jnp.full((L,), 1, dtype=jnp.int32)),
        jnp.float32,
    )
    half = x * 0.5
    for _ in range(3):
        y = y * (1.5 - half * y * y)
    return y


def _body(left_hbm, right_hbm, table_hbm, pos_l_hbm, pos_r_hbm, out_hbm,
          idx_l, idx_r, blk_l, blk_r, out_v, pos_v, sem, sem2):
    wid = lax.axis_index("s") * NC + lax.axis_index("c")
    base = wid * B_PER_W

    pltpu.sync_copy(left_hbm.at[pl.ds(base, B_PER_W)], idx_l)
    pltpu.sync_copy(right_hbm.at[pl.ds(base, B_PER_W)], idx_r)
    pltpu.sync_copy(pos_l_hbm, pos_v.at[0])
    pltpu.sync_copy(pos_r_hbm, pos_v.at[1])
    psum = [pos_v[0, pl.ds(c * L, L)] + pos_v[1, pl.ds(c * L, L)]
            for c in range(DC)]

    def fire(g, buf, s):
        vl = idx_l[pl.ds(g * L, L)]
        vr = idx_r[pl.ds(g * L, L)]
        for k in range(L):
            rl = vl[k]
            pltpu.make_async_copy(
                table_hbm.at[pl.ds((rl // 8) * 8, 8), :],
                blk_l.at[buf, k], s).start()
        for k in range(L):
            rr = vr[k]
            pltpu.make_async_copy(
                table_hbm.at[pl.ds((rr // 8) * 8, 8), :],
                blk_r.at[buf, k], s).start()

    fire(0, 0, sem)
    fire(1, 1, sem2)

    def half_grp(g, buf, s):
        pltpu.make_async_copy(
            table_hbm.at[pl.ds(0, L * 8), :], blk_l.at[buf], s).wait()
        pltpu.make_async_copy(
            table_hbm.at[pl.ds(0, L * 8), :], blk_r.at[buf], s).wait()

        vl = idx_l[pl.ds(g * L, L)]
        vr = idx_r[pl.ds(g * L, L)]
        for k in range(L):
            sl = vl[k] % 8
            sr = vr[k] % 8
            v = [blk_l[buf, k, sl, pl.ds(c * L, L)]
                 + blk_r[buf, k, sr, pl.ds(c * L, L)] + psum[c]
                 for c in range(DC)]
            ss = v[0] * v[0]
            for c in range(1, DC):
                ss = ss + v[c] * v[c]
            tot = _lane_sum16(ss)
            rinv = _rsqrt16(jnp.maximum(tot, 1e-24))
            row2 = g * (L // 2) + (k // 2)
            off = (k % 2) * D
            for c in range(DC):
                out_v[row2, pl.ds(off + c * L, L)] = v[c] * rinv

        @pl.when(g < NG - 2)
        def _():
            fire(g + 2, buf, s)

    def grp(i, carry):
        half_grp(2 * i, 0, sem)
        half_grp(2 * i + 1, 1, sem2)
        return carry

    lax.fori_loop(0, NG // 2, grp, 0)

    pltpu.sync_copy(out_v, out_hbm.at[pl.ds(wid * (B_PER_W // 2),
                                            B_PER_W // 2)])


@jax.jit
def kernel(left_idx, right_idx, class_emb, pos_left, pos_right):
    mesh = plsc.VectorSubcoreMesh(core_axis_name="c", subcore_axis_name="s")
    run = pl.kernel(
        _body,
        out_type=jax.ShapeDtypeStruct((BATCH // 2, 2 * D), jnp.float32),
        mesh=mesh,
        compiler_params=pltpu.CompilerParams(
            needs_layout_passes=False, use_tc_tiling_on_sc=True),
        scratch_types=[
            pltpu.VMEM((B_PER_W,), jnp.int32),
            pltpu.VMEM((B_PER_W,), jnp.int32),
            pltpu.VMEM((2, L, 8, D), jnp.float32),
            pltpu.VMEM((2, L, 8, D), jnp.float32),
            pltpu.VMEM((B_PER_W // 2, 2 * D), jnp.float32),
            pltpu.VMEM((2, D), jnp.float32),
            pltpu.SemaphoreType.DMA,
            pltpu.SemaphoreType.DMA,
        ],
    )
    out2 = run(left_idx.astype(jnp.int32), right_idx.astype(jnp.int32),
               class_emb, pos_left, pos_right)
    return out2.reshape(BATCH, D)


# speedup vs baseline: 11.0910x; 1.3162x over previous
import jax
import jax.numpy as jnp
from jax import lax
from jax.experimental import pallas as pl
from jax.experimental.pallas import tpu as pltpu
from jax.experimental.pallas import tpu_sc as plsc

NUM_ROWS = 1000000
D = 64
BATCH = 16384

NC = 2
NS = 16
NW = NC * NS
B_PER_W = BATCH // NW
L = 16
DC = D // L
NG = B_PER_W // L


def _lane_sum16(v):
    io = lax.iota(jnp.int32, L)
    dn = lax.GatherDimensionNumbers(
        offset_dims=(), collapsed_slice_dims=(0,), start_index_map=(0,))
    for k in (8, 4, 2, 1):
        idx = lax.bitwise_xor(io, jnp.full((L,), k, dtype=jnp.int32))
        v = v + lax.gather(v, idx[:, None], dn, (1,),
                           mode=lax.GatherScatterMode.PROMISE_IN_BOUNDS)
    return v


def _rsqrt16(x):
    i = plsc.bitcast(x, jnp.int32)
    y = plsc.bitcast(
        jnp.full((L,), 0x5F3759DF, dtype=jnp.int32)
        - lax.shift_right_logical(i, jnp.full((L,), 1, dtype=jnp.int32)),
        jnp.float32,
    )
    half = x * 0.5
    for _ in range(3):
        y = y * (1.5 - half * y * y)
    return y


def _body(left_hbm, right_hbm, table_hbm, pos_l_hbm, pos_r_hbm, out_hbm,
          idx_l, idx_r, blk_l, blk_r, out_v, pos_v, sem, sem2):
    wid = lax.axis_index("s") * NC + lax.axis_index("c")
    base = wid * B_PER_W

    pltpu.sync_copy(left_hbm.at[pl.ds(base, B_PER_W)], idx_l)
    pltpu.sync_copy(right_hbm.at[pl.ds(base, B_PER_W)], idx_r)
    pltpu.sync_copy(pos_l_hbm, pos_v.at[0])
    pltpu.sync_copy(pos_r_hbm, pos_v.at[1])
    psum = [pos_v[0, pl.ds(c * L, L)] + pos_v[1, pl.ds(c * L, L)]
            for c in range(DC)]

    def fire(g, buf, s):
        vl = idx_l[pl.ds(g * L, L)]
        vr = idx_r[pl.ds(g * L, L)]
        for k in range(L):
            rl = vl[k]
            hi = rl // 500000
            rr0 = rl - hi * 500000
            pltpu.make_async_copy(
                table_hbm.at[hi, pl.ds((rr0 // 8) * 8, 8), :],
                blk_l.at[buf, k], s).start()
        for k in range(L):
            rr = vr[k]
            hi = rr // 500000
            rr0 = rr - hi * 500000
            pltpu.make_async_copy(
                table_hbm.at[hi, pl.ds((rr0 // 8) * 8, 8), :],
                blk_r.at[buf, k], s).start()

    fire(0, 0, sem)
    fire(1, 1, sem2)

    def half_grp(g, buf, s):
        pltpu.make_async_copy(
            table_hbm.at[0, pl.ds(0, L * 8), :], blk_l.at[buf], s).wait()
        pltpu.make_async_copy(
            table_hbm.at[0, pl.ds(0, L * 8), :], blk_r.at[buf], s).wait()

        vl = idx_l[pl.ds(g * L, L)]
        vr = idx_r[pl.ds(g * L, L)]
        for k in range(L):
            sl = vl[k] % 8
            sr = vr[k] % 8
            v = [blk_l[buf, k, sl, pl.ds(c * L, L)]
                 + blk_r[buf, k, sr, pl.ds(c * L, L)] + psum[c]
                 for c in range(DC)]
            ss = v[0] * v[0]
            for c in range(1, DC):
                ss = ss + v[c] * v[c]
            tot = _lane_sum16(ss)
            rinv = _rsqrt16(jnp.maximum(tot, 1e-24))
            row2 = g * (L // 2) + (k // 2)
            off = (k % 2) * D
            for c in range(DC):
                out_v[row2, pl.ds(off + c * L, L)] = v[c] * rinv

        @pl.when(g < NG - 2)
        def _():
            fire(g + 2, buf, s)

    def grp(i, carry):
        half_grp(2 * i, 0, sem)
        half_grp(2 * i + 1, 1, sem2)
        return carry

    lax.fori_loop(0, NG // 2, grp, 0)

    pltpu.sync_copy(out_v, out_hbm.at[pl.ds(wid * (B_PER_W // 2),
                                            B_PER_W // 2)])


@jax.jit
def kernel(left_idx, right_idx, class_emb, pos_left, pos_right):
    mesh = plsc.VectorSubcoreMesh(core_axis_name="c", subcore_axis_name="s")
    run = pl.kernel(
        _body,
        out_type=jax.ShapeDtypeStruct((BATCH // 2, 2 * D), jnp.float32),
        mesh=mesh,
        compiler_params=pltpu.CompilerParams(
            needs_layout_passes=False, use_tc_tiling_on_sc=True),
        scratch_types=[
            pltpu.VMEM((B_PER_W,), jnp.int32),
            pltpu.VMEM((B_PER_W,), jnp.int32),
            pltpu.VMEM((2, L, 8, D), jnp.float32),
            pltpu.VMEM((2, L, 8, D), jnp.float32),
            pltpu.VMEM((B_PER_W // 2, 2 * D), jnp.float32),
            pltpu.VMEM((2, D), jnp.float32),
            pltpu.SemaphoreType.DMA,
            pltpu.SemaphoreType.DMA,
        ],
    )
    out2 = run(left_idx.astype(jnp.int32), right_idx.astype(jnp.int32),
               class_emb.reshape(2, NUM_ROWS // 2, D), pos_left, pos_right)
    return out2.reshape(BATCH, D)


# speedup vs baseline: 11.5884x; 1.0448x over previous
import jax
import jax.numpy as jnp
from jax import lax
from jax.experimental import pallas as pl
from jax.experimental.pallas import tpu as pltpu
from jax.experimental.pallas import tpu_sc as plsc

NUM_ROWS = 1000000
D = 64
BATCH = 16384

NC = 2
NS = 16
NW = NC * NS
B_PER_W = BATCH // NW
L = 16
DC = D // L
NG = B_PER_W // L


def _lane_sum16(v):
    io = lax.iota(jnp.int32, L)
    dn = lax.GatherDimensionNumbers(
        offset_dims=(), collapsed_slice_dims=(0,), start_index_map=(0,))
    for k in (8, 4, 2, 1):
        idx = lax.bitwise_xor(io, jnp.full((L,), k, dtype=jnp.int32))
        v = v + lax.gather(v, idx[:, None], dn, (1,),
                           mode=lax.GatherScatterMode.PROMISE_IN_BOUNDS)
    return v


def _rsqrt16(x):
    i = plsc.bitcast(x, jnp.int32)
    y = plsc.bitcast(
        jnp.full((L,), 0x5F3759DF, dtype=jnp.int32)
        - lax.shift_right_logical(i, jnp.full((L,), 1, dtype=jnp.int32)),
        jnp.float32,
    )
    half = x * 0.5
    for _ in range(3):
        y = y * (1.5 - half * y * y)
    return y


def _body(left_hbm, right_hbm, table_hbm, pos_l_hbm, pos_r_hbm, out_hbm,
          idx_l, idx_r, blk_l, blk_r, out_v, pos_v, sem, sem2):
    wid = lax.axis_index("s") * NC + lax.axis_index("c")
    base = wid * B_PER_W

    pltpu.sync_copy(left_hbm.at[pl.ds(base, B_PER_W)], idx_l)
    pltpu.sync_copy(right_hbm.at[pl.ds(base, B_PER_W)], idx_r)
    pltpu.sync_copy(pos_l_hbm, pos_v.at[0])
    pltpu.sync_copy(pos_r_hbm, pos_v.at[1])
    psum = [pos_v[0, pl.ds(c * L, L)] + pos_v[1, pl.ds(c * L, L)]
            for c in range(DC)]

    def fire(g, buf, s):
        vl = idx_l[pl.ds(g * L, L)]
        vr = idx_r[pl.ds(g * L, L)]
        half_rows = NUM_ROWS // 2
        hl = vl // half_rows
        sl = ((vl - hl * half_rows) // 8) * 8
        hr = vr // half_rows
        sr = ((vr - hr * half_rows) // 8) * 8
        for k in range(L):
            pltpu.make_async_copy(
                table_hbm.at[hl[k], pl.ds(pl.multiple_of(sl[k], 8), 8), :],
                blk_l.at[buf, k], s).start()
        for k in range(L):
            pltpu.make_async_copy(
                table_hbm.at[hr[k], pl.ds(pl.multiple_of(sr[k], 8), 8), :],
                blk_r.at[buf, k], s).start()

    fire(0, 0, sem)
    fire(1, 1, sem2)

    def half_grp(g, buf, s):
        pltpu.make_async_copy(
            table_hbm.at[0, pl.ds(0, L * 8), :], blk_l.at[buf], s).wait()
        pltpu.make_async_copy(
            table_hbm.at[0, pl.ds(0, L * 8), :], blk_r.at[buf], s).wait()

        vl = idx_l[pl.ds(g * L, L)]
        vr = idx_r[pl.ds(g * L, L)]
        for k in range(L):
            sl = vl[k] % 8
            sr = vr[k] % 8
            v = [blk_l[buf, k, sl, pl.ds(c * L, L)]
                 + blk_r[buf, k, sr, pl.ds(c * L, L)] + psum[c]
                 for c in range(DC)]
            ss = v[0] * v[0]
            for c in range(1, DC):
                ss = ss + v[c] * v[c]
            tot = _lane_sum16(ss)
            rinv = _rsqrt16(jnp.maximum(tot, 1e-24))
            row2 = g * (L // 2) + (k // 2)
            off = (k % 2) * D
            for c in range(DC):
                out_v[row2, pl.ds(off + c * L, L)] = v[c] * rinv

        @pl.when(g < NG - 2)
        def _():
            fire(g + 2, buf, s)

    def grp(i, carry):
        half_grp(2 * i, 0, sem)
        half_grp(2 * i + 1, 1, sem2)
        return carry

    lax.fori_loop(0, NG // 2, grp, 0)

    pltpu.sync_copy(out_v, out_hbm.at[pl.ds(wid * (B_PER_W // 2),
                                            B_PER_W // 2)])


@jax.jit
def kernel(left_idx, right_idx, class_emb, pos_left, pos_right):
    mesh = plsc.VectorSubcoreMesh(core_axis_name="c", subcore_axis_name="s")
    run = pl.kernel(
        _body,
        out_type=jax.ShapeDtypeStruct((BATCH // 2, 2 * D), jnp.float32),
        mesh=mesh,
        compiler_params=pltpu.CompilerParams(
            needs_layout_passes=False, use_tc_tiling_on_sc=True),
        scratch_types=[
            pltpu.VMEM((B_PER_W,), jnp.int32),
            pltpu.VMEM((B_PER_W,), jnp.int32),
            pltpu.VMEM((2, L, 8, D), jnp.float32),
            pltpu.VMEM((2, L, 8, D), jnp.float32),
            pltpu.VMEM((B_PER_W // 2, 2 * D), jnp.float32),
            pltpu.VMEM((2, D), jnp.float32),
            pltpu.SemaphoreType.DMA,
            pltpu.SemaphoreType.DMA,
        ],
    )
    out2 = run(left_idx.astype(jnp.int32), right_idx.astype(jnp.int32),
               class_emb.reshape(2, NUM_ROWS // 2, D), pos_left, pos_right)
    return out2.reshape(BATCH, D)
